# Initial kernel scaffold; baseline (speedup 1.0000x reference)
import functools

import jax
import jax.numpy as jnp
from jax import lax
from jax.experimental import pallas as pl
from jax.experimental.pallas import tpu as pltpu
from jax.experimental.pallas import tpu_sc as plsc

B, H, E, O = 16384, 50, 32, 64
N = B * H
NC, NS = 2, 16
NW = NC * NS
PER_W = N // NW
GCHUNK = 128
CHUNK = 1024
NG = CHUNK // GCHUNK
NCHUNKS = PER_W // CHUNK


def _sc_gather(idx2d, table):
    mesh = plsc.VectorSubcoreMesh(core_axis_name="c", subcore_axis_name="s")

    @functools.partial(
        pl.kernel,
        mesh=mesh,
        out_type=jax.ShapeDtypeStruct((N, E), jnp.float32),
        scratch_types=[
            pltpu.VMEM((NG, GCHUNK), jnp.int32),
            pltpu.VMEM((CHUNK, E), jnp.float32),
            pltpu.SemaphoreType.DMA,
        ],
    )
    def k(idx_hbm, table_hbm, out_hbm, idx_v, rows_v, sem):
        wid = lax.axis_index("s") * NC + lax.axis_index("c")
        base = wid * PER_W

        def body(i, carry):
            off = base + i * CHUNK
            pltpu.sync_copy(idx_hbm.at[pl.ds(off // GCHUNK, NG)], idx_v)
            copies = [
                pltpu.async_copy(
                    table_hbm.at[idx_v.at[j]],
                    rows_v.at[pl.ds(j * GCHUNK, GCHUNK)],
                    sem,
                )
                for j in range(NG)
            ]
            for cp in copies:
                cp.wait()
            pltpu.sync_copy(rows_v, out_hbm.at[pl.ds(off, CHUNK)])
            return carry

        lax.fori_loop(0, NCHUNKS, body, 0)

    return k(idx2d, table)


def _tc_linear_relu(x, wt, b2):
    blk = 2048

    def body(x_ref, w_ref, b_ref, o_ref):
        acc = jnp.dot(x_ref[...], w_ref[...], preferred_element_type=jnp.float32)
        o_ref[...] = jnp.maximum(acc + b_ref[...], 0.0)

    return pl.pallas_call(
        body,
        grid=(N // blk,),
        in_specs=[
            pl.BlockSpec((blk, E), lambda i: (i, 0)),
            pl.BlockSpec((E, O), lambda i: (0, 0)),
            pl.BlockSpec((1, O), lambda i: (0, 0)),
        ],
        out_specs=pl.BlockSpec((blk, O), lambda i: (i, 0)),
        out_shape=jax.ShapeDtypeStruct((N, O), jnp.float32),
    )(x, wt, b2)


def kernel(category, table, W, b):
    idx2d = category.astype(jnp.int32).reshape(N // GCHUNK, GCHUNK)
    gathered = _sc_gather(idx2d, table)
    out = _tc_linear_relu(gathered, W.T, b.reshape(1, O))
    return out.reshape(B, H, O)


# speedup vs baseline: 11.0013x; 11.0013x over previous
import functools

import jax
import jax.numpy as jnp
from jax import lax
from jax.experimental import pallas as pl
from jax.experimental.pallas import tpu as pltpu
from jax.experimental.pallas import tpu_sc as plsc

B, H, E, O = 16384, 50, 32, 64
N = B * H
NC, NS = 2, 16
NW = NC * NS
PER_W = N // NW
GCHUNK = 128
CHUNK = 1024
NG = CHUNK // GCHUNK
NCHUNKS = PER_W // CHUNK


def _sc_gather(idx2d, table):
    mesh = plsc.VectorSubcoreMesh(core_axis_name="c", subcore_axis_name="s")

    @functools.partial(
        pl.kernel,
        mesh=mesh,
        out_type=jax.ShapeDtypeStruct((N, E), jnp.float32),
        scratch_types=[
            pltpu.VMEM((NG, GCHUNK), jnp.int32),
            pltpu.VMEM((CHUNK, E), jnp.float32),
            pltpu.SemaphoreType.DMA,
        ],
        compiler_params=pltpu.CompilerParams(use_tc_tiling_on_sc=False),
    )
    def k(idx_hbm, table_hbm, out_hbm, idx_v, rows_v, sem):
        wid = lax.axis_index("s") * NC + lax.axis_index("c")
        base = wid * PER_W

        def body(i, carry):
            off = pl.multiple_of(base + i * CHUNK, CHUNK)
            pltpu.sync_copy(idx_hbm.at[pl.ds(pl.multiple_of(off // GCHUNK, NG), NG)], idx_v)
            copies = [
                pltpu.async_copy(
                    table_hbm.at[idx_v.at[j]],
                    rows_v.at[pl.ds(j * GCHUNK, GCHUNK)],
                    sem,
                )
                for j in range(NG)
            ]
            for cp in copies:
                cp.wait()
            pltpu.sync_copy(rows_v, out_hbm.at[pl.ds(off, CHUNK)])
            return carry

        lax.fori_loop(0, NCHUNKS, body, 0)

    return k(idx2d, table)


def _tc_linear_relu(x, wt, b2):
    blk = 2048

    def body(x_ref, w_ref, b_ref, o_ref):
        acc = jnp.dot(x_ref[...], w_ref[...], preferred_element_type=jnp.float32)
        o_ref[...] = jnp.maximum(acc + b_ref[...], 0.0)

    return pl.pallas_call(
        body,
        grid=(N // blk,),
        in_specs=[
            pl.BlockSpec((blk, E), lambda i: (i, 0)),
            pl.BlockSpec((E, O), lambda i: (0, 0)),
            pl.BlockSpec((1, O), lambda i: (0, 0)),
        ],
        out_specs=pl.BlockSpec((blk, O), lambda i: (i, 0)),
        out_shape=jax.ShapeDtypeStruct((N, O), jnp.float32),
    )(x, wt, b2)


def kernel(category, table, W, b):
    idx2d = category.astype(jnp.int32).reshape(N // GCHUNK, GCHUNK)
    gathered = _sc_gather(idx2d, table)
    out = _tc_linear_relu(gathered, W.T, b.reshape(1, O))
    return out.reshape(B, H, O)


# speedup vs baseline: 16.2748x; 1.4794x over previous
import functools

import jax
import jax.numpy as jnp
from jax import lax
from jax.experimental import pallas as pl
from jax.experimental.pallas import tpu as pltpu
from jax.experimental.pallas import tpu_sc as plsc

B, H, E, O = 16384, 50, 32, 64
N = B * H
NC, NS = 2, 16
NW = NC * NS
PER_W = N // NW
GCHUNK = 128
CHUNK = 1024
NG = CHUNK // GCHUNK
NCHUNKS = PER_W // CHUNK


def _sc_gather(idx2d, table):
    mesh = plsc.VectorSubcoreMesh(core_axis_name="c", subcore_axis_name="s")

    @functools.partial(
        pl.kernel,
        mesh=mesh,
        out_type=jax.ShapeDtypeStruct((N, E), jnp.float32),
        scratch_types=[
            pltpu.VMEM((NG, GCHUNK), jnp.int32),
            pltpu.VMEM((CHUNK, E), jnp.float32),
            pltpu.SemaphoreType.DMA,
        ],
        compiler_params=pltpu.CompilerParams(use_tc_tiling_on_sc=False),
    )
    def k(idx_hbm, table_hbm, out_hbm, idx_v, rows_v, sem):
        wid = lax.axis_index("s") * NC + lax.axis_index("c")
        base = wid * PER_W

        def body(i, carry):
            off = pl.multiple_of(base + i * CHUNK, CHUNK)
            pltpu.sync_copy(idx_hbm.at[pl.ds(pl.multiple_of(off // GCHUNK, NG), NG)], idx_v)
            copies = [
                pltpu.async_copy(
                    table_hbm.at[idx_v.at[j]],
                    rows_v.at[pl.ds(j * GCHUNK, GCHUNK)],
                    sem,
                )
                for j in range(NG)
            ]
            for cp in copies:
                cp.wait()
            pltpu.sync_copy(rows_v, out_hbm.at[pl.ds(off, CHUNK)])
            return carry

        lax.fori_loop(0, NCHUNKS, body, 0)

    return k(idx2d, table)


FOLD = 128 // E
NF = N // FOLD
OF = O * FOLD


def _tc_linear_relu(xf, wd, bf):
    blk = 1024

    def body(x_ref, w_ref, b_ref, o_ref):
        acc = jnp.dot(x_ref[...], w_ref[...], preferred_element_type=jnp.float32)
        o_ref[...] = jnp.maximum(acc + b_ref[...], 0.0)

    return pl.pallas_call(
        body,
        grid=(NF // blk,),
        in_specs=[
            pl.BlockSpec((blk, 128), lambda i: (i, 0)),
            pl.BlockSpec((128, OF), lambda i: (0, 0)),
            pl.BlockSpec((1, OF), lambda i: (0, 0)),
        ],
        out_specs=pl.BlockSpec((blk, OF), lambda i: (i, 0)),
        out_shape=jax.ShapeDtypeStruct((NF, OF), jnp.float32),
    )(xf, wd, bf)


def kernel(category, table, W, b):
    idx2d = category.astype(jnp.int32).reshape(N // GCHUNK, GCHUNK)
    gathered = _sc_gather(idx2d, table)
    xf = gathered.reshape(NF, 128)
    wd = jnp.kron(jnp.eye(FOLD, dtype=jnp.float32), W.T)
    bf = jnp.tile(b, FOLD).reshape(1, OF)
    out = _tc_linear_relu(xf, wd, bf)
    return out.reshape(B, H, O)
